# Initial kernel scaffold; baseline (speedup 1.0000x reference)
import jax
import jax.numpy as jnp
from jax.experimental import pallas as pl


def kernel(history_data, TID, DIW, node_emb, node_emb_u, node_emb_d, emb1_w, emb2_w, Wts, bts, W1a, b1a, W1b, b1b, W1c, b1c, W2a, b2a, W2b, b2b, W2c, b2c, W_fore, b_fore):
    raise NotImplementedError("write your pallas kernel here")


# speedup vs baseline: 5.9315x; 5.9315x over previous
import jax
import jax.numpy as jnp
from jax import lax
from jax.experimental import pallas as pl
from jax.experimental.pallas import tpu as pltpu

B = 32; L = 12; N = 883; C = 3
NP = 896
TOPK = 20
TOD = 288; DOW = 7; SEQ_OUT = 12
HID = 128


def _kth_largest(x, k, nbits=30):
    rows = x.shape[0]
    kf = jnp.float32(k)

    def body(_, carry):
        lo, hi = carry
        mid = lo + (hi - lo) // 2
        t = lax.bitcast_convert_type(mid, jnp.float32)
        c = jnp.sum((x >= t).astype(jnp.float32), axis=1, keepdims=True)
        ge = c >= kf
        return jnp.where(ge, mid, lo), jnp.where(ge, hi, mid)

    lo0 = jnp.zeros((rows, 1), jnp.int32)
    hi0 = jnp.full((rows, 1), 0x3F800001, jnp.int32)
    lo, _ = lax.fori_loop(0, nbits, body, (lo0, hi0))
    return lax.bitcast_convert_type(lo, jnp.float32)


def _main_body(hist_ref, tid_ref, diw_ref, ne_ref, neu_ref, ned_ref, e1_ref,
               TID_ref, DIW_ref, Wts_ref, bts_ref,
               W1a_ref, b1a_ref, W1b_ref, b1b_ref, W1c_ref, b1c_ref,
               Wf_ref, bf_ref, out_ref, static_scr):
    b = pl.program_id(0)

    @pl.when(b == 0)
    def _():
        r = lax.dot_general(ned_ref[...], neu_ref[...],
                            (((1,), (1,)), ((), ())),
                            preferred_element_type=jnp.float32)
        col = lax.broadcasted_iota(jnp.int32, (NP, NP), 1)
        valid = col < N
        r = jnp.where(valid, jnp.maximum(r, 0.0), -1e30)
        m = jnp.max(r, axis=1, keepdims=True)
        e = jnp.exp(r - m)
        sg = e / jnp.sum(e, axis=1, keepdims=True)
        thr = _kth_largest(sg, TOPK)
        gt = sg > thr
        ties = (sg == thr) & valid
        row_i = lax.broadcasted_iota(jnp.int32, (NP, NP), 0)
        tri = (row_i <= col).astype(jnp.float32)
        rank = lax.dot_general(ties.astype(jnp.float32), tri,
                               (((1,), (0,)), ((), ())),
                               preferred_element_type=jnp.float32)
        need = jnp.float32(TOPK) - jnp.sum(gt.astype(jnp.float32), axis=1,
                                           keepdims=True)
        keep = gt | (ties & (rank <= need))
        static_scr[...] = jnp.where(keep, sg, 0.0)

    ts = jnp.dot(hist_ref[0], Wts_ref[...],
                 preferred_element_type=jnp.float32) + bts_ref[...]
    oh_t = (tid_ref[0] == lax.broadcasted_iota(jnp.int32, (NP, TOD), 1))
    emb_t = jnp.dot(oh_t.astype(jnp.float32), TID_ref[...],
                    preferred_element_type=jnp.float32)
    oh_d = (diw_ref[0] == lax.broadcasted_iota(jnp.int32, (NP, 8), 1))
    emb_d = jnp.dot(oh_d.astype(jnp.float32), DIW_ref[...],
                    preferred_element_type=jnp.float32)
    H = jnp.concatenate([ts, ne_ref[...], emb_t, emb_d], axis=1)

    h1 = jnp.maximum(jnp.dot(H, W1a_ref[...],
                             preferred_element_type=jnp.float32)
                     + b1a_ref[...], 0.0)
    h2 = jnp.maximum(jnp.dot(h1, W1b_ref[...],
                             preferred_element_type=jnp.float32)
                     + b1b_ref[...], 0.0)
    f1 = jnp.dot(h2, W1c_ref[...],
                 preferred_element_type=jnp.float32) + b1c_ref[...]
    nv = jnp.tanh(e1_ref[...] * f1)

    a = lax.dot_general(nv, nv, (((1,), (1,)), ((), ())),
                        preferred_element_type=jnp.float32)
    adj = jnp.maximum(jnp.tanh(a), 0.0)
    thr = _kth_largest(adj, TOPK)
    dyn = jnp.where(adj >= thr, adj, 0.0)

    hs = jnp.dot(static_scr[...], H, preferred_element_type=jnp.float32)
    hd = jnp.dot(dyn, H, preferred_element_type=jnp.float32)
    fused = jnp.maximum(hs + hd + H, 0.0)
    out_ref[0] = jnp.dot(fused, Wf_ref[...],
                         preferred_element_type=jnp.float32) + bf_ref[...]


def kernel(history_data, TID, DIW, node_emb, node_emb_u, node_emb_d,
           emb1_w, emb2_w, Wts, bts, W1a, b1a, W1b, b1b, W1c, b1c,
           W2a, b2a, W2b, b2b, W2c, b2c, W_fore, b_fore):
    f32 = jnp.float32
    tid_idx = (history_data[:, -1, :, 1] * TOD).astype(jnp.int32)
    diw_idx = (history_data[:, -1, :, 2] * DOW).astype(jnp.int32)
    pad_n = NP - N
    tid_p = jnp.pad(tid_idx, ((0, 0), (0, pad_n)))[..., None]
    diw_p = jnp.pad(diw_idx, ((0, 0), (0, pad_n)))[..., None]
    hist2 = history_data.transpose(0, 2, 1, 3).reshape(B, N, L * C)
    hist2 = jnp.pad(hist2, ((0, 0), (0, pad_n), (0, 0)))
    ne_p = jnp.pad(node_emb, ((0, pad_n), (0, 0)))
    neu_p = jnp.pad(node_emb_u, ((0, pad_n), (0, 0)))
    ned_p = jnp.pad(node_emb_d, ((0, pad_n), (0, 0)))
    e1_p = jnp.pad(emb1_w, ((0, pad_n), (0, 0)))
    DIW8 = jnp.pad(DIW, ((0, 1), (0, 0)))

    full = lambda shape: pl.BlockSpec(shape, lambda b: (0,) * len(shape))
    perb2 = lambda shape: pl.BlockSpec((1,) + shape, lambda b: (b, 0, 0))

    out = pl.pallas_call(
        _main_body,
        grid=(B,),
        in_specs=[
            perb2((NP, L * C)),
            perb2((NP, 1)),
            perb2((NP, 1)),
            full((NP, 32)),
            full((NP, 32)),
            full((NP, 32)),
            full((NP, 40)),
            full((TOD, 32)),
            full((8, 32)),
            full((L * C, 32)),
            full((1, 32)),
            full((HID, 64)),
            full((1, 64)),
            full((64, 64)),
            full((1, 64)),
            full((64, 40)),
            full((1, 40)),
            full((HID, SEQ_OUT)),
            full((1, SEQ_OUT)),
        ],
        out_specs=perb2((NP, SEQ_OUT)),
        out_shape=jax.ShapeDtypeStruct((B, NP, SEQ_OUT), f32),
        scratch_shapes=[pltpu.VMEM((NP, NP), f32)],
    )(hist2, tid_p, diw_p, ne_p, neu_p, ned_p, e1_p, TID, DIW8,
      Wts, bts[None, :], W1a, b1a[None, :], W1b, b1b[None, :],
      W1c, b1c[None, :], W_fore, b_fore[None, :])
    return out[:, :N, :]
